# baseline (speedup 1.0000x reference)
import numpy as np
import jax
import jax.numpy as jnp
from jax import lax
from jax.experimental import pallas as pl
from jax.experimental.pallas import tpu as pltpu
from jax.experimental.pallas import tpu_sc as plsc

N = 65536
D = 64
K = int(N * 0.1)

NUM_CORES = 2
NUM_SUBCORES = 16
LANES = 16
NWORKERS = NUM_CORES * NUM_SUBCORES
COLS_PER_W = D // NWORKERS

NVEC = N // LANES
MIN32 = np.int32(-(2**31))


def _sc_select_body(bits_hbm, out_hbm, colbuf, hist, persum, rowbuf, shared,
                    sem0, sem1):
  wid = lax.axis_index("s") * NUM_CORES + lax.axis_index("c")
  lane = lax.iota(jnp.int32, LANES)
  ones = jnp.ones((LANES,), jnp.int32)
  zeros16 = jnp.zeros((LANES,), jnp.int32)

  def extract(vec, idx):
    return jnp.sum(jnp.where(lane == idx, vec, 0))

  def desc_scan(get, n16, r):
    def body(jj, c):
      acc, idx, rn, cnt, done = c
      j = n16 - 1 - jj
      vec = get(j)
      rv = lax.rev(vec, (0,))
      cs = jnp.cumsum(rv)
      cond = (acc + cs) >= r
      lam = jnp.sum(jnp.where(cond, 0, 1))
      hit = jnp.logical_and(lam < LANES, done == 0)
      csl = extract(cs, lam)
      rvl = extract(rv, lam)
      idx = jnp.where(hit, j * LANES + (LANES - 1) - lam, idx)
      rn = jnp.where(hit, r - (acc + csl - rvl), rn)
      cnt = jnp.where(hit, rvl, cnt)
      done = jnp.where(hit, 1, done)
      return acc + jnp.sum(vec), idx, rn, cnt, done
    z = jnp.int32(0)
    _, idx, rn, cnt, _ = lax.fori_loop(0, n16, body, (z, z, z, z, z))
    return idx, rn, cnt

  def asc_scan(get, n16, r):
    def body(j, c):
      acc, idx, rn, cnt, done = c
      vec = get(j)
      cs = jnp.cumsum(vec)
      cond = (acc + cs) >= r
      lam = jnp.sum(jnp.where(cond, 0, 1))
      hit = jnp.logical_and(lam < LANES, done == 0)
      csl = extract(cs, lam)
      vl = extract(vec, lam)
      idx = jnp.where(hit, j * LANES + lam, idx)
      rn = jnp.where(hit, r - (acc + csl - vl), rn)
      cnt = jnp.where(hit, vl, cnt)
      done = jnp.where(hit, 1, done)
      return acc + jnp.sum(vec), idx, rn, cnt, done
    z = jnp.int32(0)
    _, idx, rn, cnt, _ = lax.fori_loop(0, n16, body, (z, z, z, z, z))
    return idx, rn, cnt

  def build_persum(nbins):
    @plsc.parallel_loop(0, nbins // LANES, unroll=8)
    def _(j):
      s = jnp.sum(hist[pl.ds(j * LANES, LANES)])
      plsc.store_scatter(persum, [jnp.full((LANES,), j, jnp.int32)],
                         jnp.full((LANES,), s, jnp.int32), mask=lane == 0)

  def clear_hist(nwords):
    @plsc.parallel_loop(0, nwords // LANES, unroll=8)
    def _(j):
      hist[pl.ds(j * LANES, LANES)] = zeros16

  def walk_desc(off, nbins, psoff, r):
    nch = nbins // LANES
    c, rc, _ = desc_scan(
        lambda j: persum[pl.ds(psoff + j * LANES, LANES)], nch // LANES, r)
    b, rn, cnt = desc_scan(
        lambda j: hist[pl.ds(off + c * LANES, LANES)], 1, rc)
    return c * LANES + b, rn, cnt

  def walk_asc(off, nbins, psoff, r):
    nch = nbins // LANES
    c, rc, _ = asc_scan(
        lambda j: persum[pl.ds(psoff + j * LANES, LANES)], nch // LANES, r)
    b, rn, cnt = asc_scan(
        lambda j: hist[pl.ds(off + c * LANES, LANES)], 1, rc)
    return c * LANES + b, rn, cnt

  def scan1_half(g):
    @plsc.parallel_loop(g * (NVEC // 2), (g + 1) * (NVEC // 2), unroll=8)
    def _(i):
      sl = pl.ds(i * LANES, LANES)
      b = colbuf[sl]
      s = lax.shift_right_arithmetic(b, 31)
      u = b ^ (s | MIN32)
      colbuf[sl] = u
      plsc.addupdate_scatter(hist, [lax.shift_right_logical(u, 19)], ones)

  def finish_column(col):

    build_persum(8192)
    p1t, r1t, _ = walk_desc(0, 8192, 0, jnp.int32(K))
    p1b, r1b, _ = walk_asc(0, 8192, 0, jnp.int32(K))
    clear_hist(4096)

    @plsc.parallel_loop(0, NVEC, unroll=8)
    def _(i):
      u = colbuf[pl.ds(i * LANES, LANES)]
      pref = lax.shift_right_logical(u, 19)
      bb = lax.shift_right_logical(u, 8) & 2047
      plsc.addupdate_scatter(hist, [bb], ones, mask=pref == p1t)
      plsc.addupdate_scatter(hist, [bb + 2048], ones, mask=pref == p1b)

    build_persum(4096)
    bt, r2t, _ = walk_desc(0, 2048, 0, r1t)
    bb_, r2b, _ = walk_asc(2048, 2048, 128, r1b)
    p2t = p1t * 2048 + bt
    p2b = p1b * 2048 + bb_
    clear_hist(512)

    @plsc.parallel_loop(0, NVEC, unroll=8)
    def _(i):
      u = colbuf[pl.ds(i * LANES, LANES)]
      pref = lax.shift_right_logical(u, 8)
      bb = u & 255
      plsc.addupdate_scatter(hist, [bb], ones, mask=pref == p2t)
      plsc.addupdate_scatter(hist, [bb + 256], ones, mask=pref == p2b)

    build_persum(512)
    bt, r3t, n_t = walk_desc(0, 256, 0, r2t)
    bb_, r3b, n_b = walk_asc(256, 256, 16, r2b)
    ut = lax.shift_left(p2t, 8) | bt
    ub = lax.shift_left(p2b, 8) | bb_

    a_t = jnp.full((LANES,), r3t, jnp.int32).astype(jnp.float32) / \
        jnp.full((LANES,), n_t, jnp.int32).astype(jnp.float32)
    a_b = jnp.full((LANES,), r3b, jnp.int32).astype(jnp.float32) / \
        jnp.full((LANES,), n_b, jnp.int32).astype(jnp.float32)

    row = jnp.where(lane == 0, ut ^ MIN32, jnp.where(lane == 1, ub ^ MIN32, 0))
    row = jnp.where(lane == 2, plsc.bitcast(a_t, jnp.int32), row)
    row = jnp.where(lane == 3, plsc.bitcast(a_b, jnp.int32), row)
    rowbuf[...] = row
    pltpu.sync_copy(rowbuf, out_hbm.at[col])

  sid = lax.axis_index("s")
  core = lax.axis_index("c")
  col_a = core * (D // NUM_CORES) + sid
  col_b = col_a + NUM_SUBCORES
  NH = N // 2

  def stage(col, half, sem):
    return pltpu.async_copy(bits_hbm.at[col, pl.ds(half * NH, NH)],
                            shared.at[sid], sem)

  def pull(half):
    pltpu.sync_copy(shared.at[sid], colbuf.at[pl.ds(half * NH, NH)])

  cp = stage(col_a, 0, sem0)
  clear_hist(8192)
  cp.wait()
  pull(0)
  cp = stage(col_a, 1, sem1)
  scan1_half(0)
  cp.wait()
  pull(1)
  cp = stage(col_b, 0, sem0)
  scan1_half(1)
  finish_column(col_a)
  clear_hist(8192)
  cp.wait()
  pull(0)
  cp = stage(col_b, 1, sem1)
  scan1_half(0)
  cp.wait()
  pull(1)
  scan1_half(1)
  finish_column(col_b)


def _sc_select(bits_t):
  mesh = plsc.VectorSubcoreMesh(core_axis_name="c", subcore_axis_name="s")
  return pl.kernel(
      _sc_select_body,
      out_type=jax.ShapeDtypeStruct((D, LANES), jnp.int32),
      mesh=mesh,
      compiler_params=pltpu.CompilerParams(needs_layout_passes=False),
      scratch_types=[
          pltpu.VMEM((N,), jnp.int32),
          pltpu.VMEM((8192,), jnp.int32),
          pltpu.VMEM((512,), jnp.int32),
          pltpu.VMEM((LANES,), jnp.int32),
          pltpu.VMEM_SHARED((NUM_SUBCORES, N // 2), jnp.int32),
          pltpu.SemaphoreType.DMA,
          pltpu.SemaphoreType.DMA,
      ],
  )(bits_t)


CHUNK = 8192
NBLK = N // CHUNK


def _tc_reduce_body(thr_ref, b_ref, t_ref, o_ref, acc_ref):
  i = pl.program_id(0)

  @pl.when(i == 0)
  def _():
    acc_ref[...] = jnp.zeros((D, CHUNK), jnp.float32)

  b = b_ref[...]
  m = lax.shift_right_logical(lax.shift_right_arithmetic(b, 31), 1)
  v = b ^ m
  tgt = t_ref[...]
  t_t = thr_ref[:, 0:1]
  t_b = thr_ref[:, 1:2]
  a_t = lax.bitcast_convert_type(thr_ref[:, 2:3], jnp.float32)
  a_b = lax.bitcast_convert_type(thr_ref[:, 3:4], jnp.float32)
  w = jnp.where(v > t_t, 1.0, jnp.where(v == t_t, a_t, 0.0)) - \
      jnp.where(v < t_b, 1.0, jnp.where(v == t_b, a_b, 0.0))
  acc_ref[...] += w * tgt

  @pl.when(i == NBLK - 1)
  def _():
    s = jnp.sum(acc_ref[...], axis=1, keepdims=True) * jnp.float32(1.0 / K)
    o_ref[...] = jnp.concatenate(
        [s, jnp.zeros((D, 7), jnp.float32)], axis=1)


def _tc_reduce(thr, bits_t, targets_t):
  return pl.pallas_call(
      _tc_reduce_body,
      grid=(NBLK,),
      in_specs=[
          pl.BlockSpec((D, 16), lambda i: (0, 0)),
          pl.BlockSpec((D, CHUNK), lambda i: (0, i)),
          pl.BlockSpec((D, CHUNK), lambda i: (0, i)),
      ],
      out_specs=pl.BlockSpec((D, 8), lambda i: (0, 0)),
      out_shape=jax.ShapeDtypeStruct((D, 8), jnp.float32),
      scratch_shapes=[pltpu.VMEM((D, CHUNK), jnp.float32)],
  )(thr, bits_t, targets_t)


@jax.jit
def kernel(preds, targets):
  bits_t = lax.bitcast_convert_type(preds, jnp.int32).T
  thr = _sc_select(bits_t)
  out = _tc_reduce(thr, bits_t, targets.T)
  return out[:, 0]


# speedup vs baseline: 1.0283x; 1.0283x over previous
import numpy as np
import jax
import jax.numpy as jnp
from jax import lax
from jax.experimental import pallas as pl
from jax.experimental.pallas import tpu as pltpu
from jax.experimental.pallas import tpu_sc as plsc

N = 65536
D = 64
K = int(N * 0.1)

NUM_CORES = 2
NUM_SUBCORES = 16
LANES = 16
NWORKERS = NUM_CORES * NUM_SUBCORES
COLS_PER_W = D // NWORKERS

NVEC = N // LANES
MIN32 = np.int32(-(2**31))


NSEG = 4
SEG = N // NSEG


def _sc_select_body(bits_hbm, out_hbm, colbuf, hist, persum, rowbuf,
                    sem0, sem1, sem2, sem3):
  wid = lax.axis_index("s") * NUM_CORES + lax.axis_index("c")
  lane = lax.iota(jnp.int32, LANES)
  ones = jnp.ones((LANES,), jnp.int32)
  zeros16 = jnp.zeros((LANES,), jnp.int32)

  def extract(vec, idx):
    return jnp.sum(jnp.where(lane == idx, vec, 0))

  def desc_scan(get, n16, r):
    def body(jj, c):
      acc, idx, rn, cnt, done = c
      j = n16 - 1 - jj
      vec = get(j)
      rv = lax.rev(vec, (0,))
      cs = jnp.cumsum(rv)
      cond = (acc + cs) >= r
      lam = jnp.sum(jnp.where(cond, 0, 1))
      hit = jnp.logical_and(lam < LANES, done == 0)
      csl = extract(cs, lam)
      rvl = extract(rv, lam)
      idx = jnp.where(hit, j * LANES + (LANES - 1) - lam, idx)
      rn = jnp.where(hit, r - (acc + csl - rvl), rn)
      cnt = jnp.where(hit, rvl, cnt)
      done = jnp.where(hit, 1, done)
      return acc + jnp.sum(vec), idx, rn, cnt, done
    z = jnp.int32(0)
    _, idx, rn, cnt, _ = lax.fori_loop(0, n16, body, (z, z, z, z, z))
    return idx, rn, cnt

  def asc_scan(get, n16, r):
    def body(j, c):
      acc, idx, rn, cnt, done = c
      vec = get(j)
      cs = jnp.cumsum(vec)
      cond = (acc + cs) >= r
      lam = jnp.sum(jnp.where(cond, 0, 1))
      hit = jnp.logical_and(lam < LANES, done == 0)
      csl = extract(cs, lam)
      vl = extract(vec, lam)
      idx = jnp.where(hit, j * LANES + lam, idx)
      rn = jnp.where(hit, r - (acc + csl - vl), rn)
      cnt = jnp.where(hit, vl, cnt)
      done = jnp.where(hit, 1, done)
      return acc + jnp.sum(vec), idx, rn, cnt, done
    z = jnp.int32(0)
    _, idx, rn, cnt, _ = lax.fori_loop(0, n16, body, (z, z, z, z, z))
    return idx, rn, cnt

  def build_persum(nbins):
    @plsc.parallel_loop(0, nbins // LANES, unroll=8)
    def _(j):
      s = jnp.sum(hist[pl.ds(j * LANES, LANES)])
      plsc.store_scatter(persum, [jnp.full((LANES,), j, jnp.int32)],
                         jnp.full((LANES,), s, jnp.int32), mask=lane == 0)

  def clear_hist(nwords):
    @plsc.parallel_loop(0, nwords // LANES, unroll=8)
    def _(j):
      hist[pl.ds(j * LANES, LANES)] = zeros16

  def walk_desc(off, nbins, psoff, r):
    nch = nbins // LANES
    c, rc, _ = desc_scan(
        lambda j: persum[pl.ds(psoff + j * LANES, LANES)], nch // LANES, r)
    b, rn, cnt = desc_scan(
        lambda j: hist[pl.ds(off + c * LANES, LANES)], 1, rc)
    return c * LANES + b, rn, cnt

  def walk_asc(off, nbins, psoff, r):
    nch = nbins // LANES
    c, rc, _ = asc_scan(
        lambda j: persum[pl.ds(psoff + j * LANES, LANES)], nch // LANES, r)
    b, rn, cnt = asc_scan(
        lambda j: hist[pl.ds(off + c * LANES, LANES)], 1, rc)
    return c * LANES + b, rn, cnt

  def per_column(jcol, carry):
    col = wid * COLS_PER_W + jcol
    sems = [sem0, sem1, sem2, sem3]
    copies = [
        pltpu.async_copy(bits_hbm.at[col, pl.ds(g * SEG, SEG)],
                         colbuf.at[pl.ds(g * SEG, SEG)], sems[g])
        for g in range(NSEG)
    ]
    clear_hist(8192)

    for g in range(NSEG):
      copies[g].wait()

      @plsc.parallel_loop(g * (NVEC // NSEG), (g + 1) * (NVEC // NSEG),
                          unroll=8)
      def _(i):
        sl = pl.ds(i * LANES, LANES)
        b = colbuf[sl]
        s = lax.shift_right_arithmetic(b, 31)
        u = b ^ (s | MIN32)
        colbuf[sl] = u
        plsc.addupdate_scatter(hist, [lax.shift_right_logical(u, 19)], ones)

    build_persum(8192)
    p1t, r1t, _ = walk_desc(0, 8192, 0, jnp.int32(K))
    p1b, r1b, _ = walk_asc(0, 8192, 0, jnp.int32(K))
    clear_hist(4096)

    @plsc.parallel_loop(0, NVEC, unroll=8)
    def _(i):
      u = colbuf[pl.ds(i * LANES, LANES)]
      pref = lax.shift_right_logical(u, 19)
      bb = lax.shift_right_logical(u, 8) & 2047
      plsc.addupdate_scatter(hist, [bb], ones, mask=pref == p1t)
      plsc.addupdate_scatter(hist, [bb + 2048], ones, mask=pref == p1b)

    build_persum(4096)
    bt, r2t, _ = walk_desc(0, 2048, 0, r1t)
    bb_, r2b, _ = walk_asc(2048, 2048, 128, r1b)
    p2t = p1t * 2048 + bt
    p2b = p1b * 2048 + bb_
    clear_hist(512)

    @plsc.parallel_loop(0, NVEC, unroll=8)
    def _(i):
      u = colbuf[pl.ds(i * LANES, LANES)]
      pref = lax.shift_right_logical(u, 8)
      bb = u & 255
      plsc.addupdate_scatter(hist, [bb], ones, mask=pref == p2t)
      plsc.addupdate_scatter(hist, [bb + 256], ones, mask=pref == p2b)

    build_persum(512)
    bt, r3t, n_t = walk_desc(0, 256, 0, r2t)
    bb_, r3b, n_b = walk_asc(256, 256, 16, r2b)
    ut = lax.shift_left(p2t, 8) | bt
    ub = lax.shift_left(p2b, 8) | bb_

    a_t = jnp.full((LANES,), r3t, jnp.int32).astype(jnp.float32) / \
        jnp.full((LANES,), n_t, jnp.int32).astype(jnp.float32)
    a_b = jnp.full((LANES,), r3b, jnp.int32).astype(jnp.float32) / \
        jnp.full((LANES,), n_b, jnp.int32).astype(jnp.float32)

    row = jnp.where(lane == 0, ut ^ MIN32, jnp.where(lane == 1, ub ^ MIN32, 0))
    row = jnp.where(lane == 2, plsc.bitcast(a_t, jnp.int32), row)
    row = jnp.where(lane == 3, plsc.bitcast(a_b, jnp.int32), row)
    rowbuf[...] = row
    pltpu.sync_copy(rowbuf, out_hbm.at[col])
    return carry

  lax.fori_loop(0, COLS_PER_W, per_column, 0)


def _sc_select(bits_t):
  mesh = plsc.VectorSubcoreMesh(core_axis_name="c", subcore_axis_name="s")
  return pl.kernel(
      _sc_select_body,
      out_type=jax.ShapeDtypeStruct((D, LANES), jnp.int32),
      mesh=mesh,
      compiler_params=pltpu.CompilerParams(needs_layout_passes=False),
      scratch_types=[
          pltpu.VMEM((N,), jnp.int32),
          pltpu.VMEM((8192,), jnp.int32),
          pltpu.VMEM((512,), jnp.int32),
          pltpu.VMEM((LANES,), jnp.int32),
          pltpu.SemaphoreType.DMA,
          pltpu.SemaphoreType.DMA,
          pltpu.SemaphoreType.DMA,
          pltpu.SemaphoreType.DMA,
      ],
  )(bits_t)


CHUNK = 8192
NBLK = N // CHUNK


def _tc_reduce_body(thr_ref, b_ref, t_ref, o_ref, acc_ref):
  i = pl.program_id(0)

  @pl.when(i == 0)
  def _():
    acc_ref[...] = jnp.zeros((D, CHUNK), jnp.float32)

  b = b_ref[...]
  m = lax.shift_right_logical(lax.shift_right_arithmetic(b, 31), 1)
  v = b ^ m
  tgt = t_ref[...]
  t_t = thr_ref[:, 0:1]
  t_b = thr_ref[:, 1:2]
  a_t = lax.bitcast_convert_type(thr_ref[:, 2:3], jnp.float32)
  a_b = lax.bitcast_convert_type(thr_ref[:, 3:4], jnp.float32)
  w = jnp.where(v > t_t, 1.0, jnp.where(v == t_t, a_t, 0.0)) - \
      jnp.where(v < t_b, 1.0, jnp.where(v == t_b, a_b, 0.0))
  acc_ref[...] += w * tgt

  @pl.when(i == NBLK - 1)
  def _():
    s = jnp.sum(acc_ref[...], axis=1, keepdims=True) * jnp.float32(1.0 / K)
    o_ref[...] = jnp.concatenate(
        [s, jnp.zeros((D, 7), jnp.float32)], axis=1)


def _tc_reduce(thr, bits_t, targets_t):
  return pl.pallas_call(
      _tc_reduce_body,
      grid=(NBLK,),
      in_specs=[
          pl.BlockSpec((D, 16), lambda i: (0, 0)),
          pl.BlockSpec((D, CHUNK), lambda i: (0, i)),
          pl.BlockSpec((D, CHUNK), lambda i: (0, i)),
      ],
      out_specs=pl.BlockSpec((D, 8), lambda i: (0, 0)),
      out_shape=jax.ShapeDtypeStruct((D, 8), jnp.float32),
      scratch_shapes=[pltpu.VMEM((D, CHUNK), jnp.float32)],
  )(thr, bits_t, targets_t)


@jax.jit
def kernel(preds, targets):
  bits_t = lax.bitcast_convert_type(preds, jnp.int32).T
  thr = _sc_select(bits_t)
  out = _tc_reduce(thr, bits_t, targets.T)
  return out[:, 0]
